# Initial kernel scaffold; baseline (speedup 1.0000x reference)
import jax
import jax.numpy as jnp
from jax.experimental import pallas as pl


def kernel(x, center_xyz, W1, b1, W2, b2):
    raise NotImplementedError("write your pallas kernel here")


# speedup vs baseline: 8.0562x; 8.0562x over previous
import functools

import jax
import jax.numpy as jnp
from jax import lax
from jax.experimental import pallas as pl
from jax.experimental.pallas import tpu as pltpu
from jax.experimental.pallas import tpu_sc as plsc

B, N, C, K, H = 8, 2048, 128, 20, 256
BN = B * N


def _yz_body(x_ref, w1_ref, b1_ref, yz_ref):
  xb = x_ref[0]
  w1a = w1_ref[:C, :]
  w1b = w1_ref[C:, :]
  wcat = jnp.concatenate([w1a, w1b - w1a], axis=1)
  yz = jnp.dot(xb, wcat, preferred_element_type=jnp.float32)
  bias = jnp.concatenate(
      [jnp.zeros((1, H), jnp.float32), b1_ref[:]], axis=1)
  yz_ref[0] = yz + bias


def _yz_call(x, w1, b1_2d, interpret=False):
  return pl.pallas_call(
      _yz_body,
      grid=(B,),
      in_specs=[
          pl.BlockSpec((1, N, C), lambda b: (b, 0, 0)),
          pl.BlockSpec((2 * C, H), lambda b: (0, 0)),
          pl.BlockSpec((1, H), lambda b: (0, 0)),
      ],
      out_specs=pl.BlockSpec((1, N, 2 * H), lambda b: (b, 0, 0)),
      out_shape=jax.ShapeDtypeStruct((B, N, 2 * H), jnp.float32),
      interpret=interpret,
  )(x, w1, b1_2d)


TN = 256


def _knn_body(xyz_ref, pt_ref, idx_ref):
  b = pl.program_id(0)
  q = xyz_ref[0]
  p = pt_ref[0]
  dot = jnp.dot(q.astype(jnp.bfloat16), p.astype(jnp.bfloat16),
                preferred_element_type=jnp.float32)
  inner = -2.0 * dot
  xxq = (q[:, 0:1] * q[:, 0:1] + q[:, 1:2] * q[:, 1:2]
         + q[:, 2:3] * q[:, 2:3])
  xxp = (p[0:1, :] * p[0:1, :] + p[1:2, :] * p[1:2, :]
         + p[2:3, :] * p[2:3, :])
  acc = xxp - ((-xxq) - inner)
  iota = lax.broadcasted_iota(jnp.int32, (TN, N), 1)
  sels = []
  for _ in range(K):
    m = jnp.min(acc, axis=1, keepdims=True)
    eq = acc == m
    sel = jnp.min(jnp.where(eq, iota, N), axis=1, keepdims=True)
    sels.append(sel)
    acc = jnp.where(iota == sel, jnp.float32(jnp.inf), acc)
  idx_ref[0] = jnp.concatenate(sels, axis=1) + b * N


def _knn_call(xyz, pt, interpret=False):
  return pl.pallas_call(
      _knn_body,
      grid=(B, N // TN),
      in_specs=[
          pl.BlockSpec((1, TN, 3), lambda b, t: (b, t, 0)),
          pl.BlockSpec((1, 3, N), lambda b, t: (b, 0, 0)),
      ],
      out_specs=pl.BlockSpec((1, TN, K), lambda b, t: (b, t, 0)),
      out_shape=jax.ShapeDtypeStruct((B, N, K), jnp.int32),
      interpret=interpret,
  )(xyz, pt)


_NC = 2
_NS = 16
_NW = _NC * _NS
_P = 4
_PK = _P * K
_NP = BN // _NW
_CHUNKS = _NP // _P


def _sc_body(y_ref, idx_ref, ymin_ref, ymax_ref,
             idx_v, rows_a, rows_b, omin_v, omax_v, sem_a, sem_b):
  cid = lax.axis_index("c")
  sid = lax.axis_index("s")
  wid = sid * _NC + cid
  crow0 = wid * _CHUNKS
  pt0 = wid * _NP

  pltpu.sync_copy(idx_ref.at[pl.ds(crow0, _CHUNKS)], idx_v)

  def gather(c, rows_v, sem):
    return pltpu.async_copy(y_ref.at[idx_v.at[c]], rows_v, sem)

  def compute(rows_v, c):
    for p in range(_P):
      r0 = p * K

      def jbody(j, _):
        js = pl.ds(j * 16, 16)
        vmin = rows_v[r0, js]
        vmax = vmin
        for k in range(1, K):
          v = rows_v[r0 + k, js]
          vmin = jnp.minimum(vmin, v)
          vmax = jnp.maximum(vmax, v)
        omin_v[p, js] = vmin
        omax_v[p, js] = vmax
        return 0

      lax.fori_loop(0, H // 16, jbody, 0)
    dst = pl.ds(pt0 + c * _P, _P)
    pltpu.sync_copy(omin_v, ymin_ref.at[dst])
    pltpu.sync_copy(omax_v, ymax_ref.at[dst])

  gather(0, rows_a, sem_a)

  def loop_body(i, _):
    c = i * 2
    gather(c + 1, rows_b, sem_b)
    pltpu.make_async_copy(y_ref.at[idx_v.at[c]], rows_a, sem_a).wait()
    compute(rows_a, c)
    gather_next = c + 2
    gather(gather_next, rows_a, sem_a)
    pltpu.make_async_copy(y_ref.at[idx_v.at[c + 1]], rows_b, sem_b).wait()
    compute(rows_b, c + 1)
    return 0

  lax.fori_loop(0, _CHUNKS // 2 - 1, loop_body, 0)
  c = _CHUNKS - 2
  gather(c + 1, rows_b, sem_b)
  pltpu.make_async_copy(y_ref.at[idx_v.at[c]], rows_a, sem_a).wait()
  compute(rows_a, c)
  pltpu.make_async_copy(y_ref.at[idx_v.at[c + 1]], rows_b, sem_b).wait()
  compute(rows_b, c + 1)


def _minmax_sc(y2, idx2):
  mesh = plsc.VectorSubcoreMesh(core_axis_name="c", subcore_axis_name="s")
  f = pl.kernel(
      _sc_body,
      out_type=(
          jax.ShapeDtypeStruct((BN, H), jnp.float32),
          jax.ShapeDtypeStruct((BN, H), jnp.float32),
      ),
      mesh=mesh,
      scratch_types=[
          pltpu.VMEM((_CHUNKS, _PK), jnp.int32),
          pltpu.VMEM((_PK, H), jnp.float32),
          pltpu.VMEM((_PK, H), jnp.float32),
          pltpu.VMEM((_P, H), jnp.float32),
          pltpu.VMEM((_P, H), jnp.float32),
          pltpu.SemaphoreType.DMA,
          pltpu.SemaphoreType.DMA,
      ],
  )
  return f(y2, idx2)


TD = 1024


def _gelu(v):
  return 0.5 * v * (1.0 + lax.erf(v * 0.7071067811865476))


def _out_body(ymin_ref, ymax_ref, z_ref, w2_ref, b2_ref, out_ref):
  zv = z_ref[:]
  h = jnp.maximum(_gelu(ymin_ref[:] + zv), _gelu(ymax_ref[:] + zv))
  out_ref[:] = (
      jnp.dot(h, w2_ref[:], preferred_element_type=jnp.float32) + b2_ref[:])


def _out_call(ymin, ymax, z, w2, b2_2d, interpret=False):
  return pl.pallas_call(
      _out_body,
      grid=(BN // TD,),
      in_specs=[
          pl.BlockSpec((TD, H), lambda t: (t, 0)),
          pl.BlockSpec((TD, H), lambda t: (t, 0)),
          pl.BlockSpec((TD, H), lambda t: (t, 0)),
          pl.BlockSpec((H, C), lambda t: (0, 0)),
          pl.BlockSpec((1, C), lambda t: (0, 0)),
      ],
      out_specs=pl.BlockSpec((TD, C), lambda t: (t, 0)),
      out_shape=jax.ShapeDtypeStruct((BN, C), jnp.float32),
      interpret=interpret,
  )(ymin, ymax, z, w2, b2_2d)


@jax.jit
def kernel(x, center_xyz, W1, b1, W2, b2):
  pt = jnp.swapaxes(center_xyz, 1, 2)
  yz = _yz_call(x, W1, jnp.reshape(b1, (1, H)))
  idx = _knn_call(center_xyz, pt)
  y2 = jnp.reshape(yz[:, :, :H], (BN, H))
  z2 = jnp.reshape(yz[:, :, H:], (BN, H))
  idx2 = jnp.reshape(idx, (BN // _P, _PK))
  ymin, ymax = _minmax_sc(y2, idx2)
  out = _out_call(ymin, ymax, z2, W2, jnp.reshape(b2, (1, C)))
  return jnp.reshape(out, (B, N, C))


# speedup vs baseline: 8.2225x; 1.0206x over previous
import functools

import jax
import jax.numpy as jnp
from jax import lax
from jax.experimental import pallas as pl
from jax.experimental.pallas import tpu as pltpu
from jax.experimental.pallas import tpu_sc as plsc

B, N, C, K, H = 8, 2048, 128, 20, 256
BN = B * N


def _yz_body(x_ref, w1_ref, b1_ref, y_ref, z_ref):
  xb = x_ref[0]
  w1a = w1_ref[:C, :]
  w1b = w1_ref[C:, :]
  wcat = jnp.concatenate([w1a, w1b - w1a], axis=1)
  yz = jnp.dot(xb, wcat, preferred_element_type=jnp.float32)
  y_ref[:] = yz[:, :H]
  z_ref[:] = yz[:, H:] + b1_ref[:]


def _yz_call(x, w1, b1_2d, interpret=False):
  return pl.pallas_call(
      _yz_body,
      grid=(B,),
      in_specs=[
          pl.BlockSpec((1, N, C), lambda b: (b, 0, 0)),
          pl.BlockSpec((2 * C, H), lambda b: (0, 0)),
          pl.BlockSpec((1, H), lambda b: (0, 0)),
      ],
      out_specs=[
          pl.BlockSpec((N, H), lambda b: (b, 0)),
          pl.BlockSpec((N, H), lambda b: (b, 0)),
      ],
      out_shape=(
          jax.ShapeDtypeStruct((BN, H), jnp.float32),
          jax.ShapeDtypeStruct((BN, H), jnp.float32),
      ),
      interpret=interpret,
  )(x, w1, b1_2d)


TN = 256


def _knn_body(xyz_ref, pt_ref, idx_ref):
  b = pl.program_id(0)
  q = xyz_ref[0]
  p = pt_ref[0]
  dot = jnp.dot(q.astype(jnp.bfloat16), p.astype(jnp.bfloat16),
                preferred_element_type=jnp.float32)
  inner = -2.0 * dot
  xxq = (q[:, 0:1] * q[:, 0:1] + q[:, 1:2] * q[:, 1:2]
         + q[:, 2:3] * q[:, 2:3])
  xxp = (p[0:1, :] * p[0:1, :] + p[1:2, :] * p[1:2, :]
         + p[2:3, :] * p[2:3, :])
  acc = xxp - ((-xxq) - inner)
  iota = lax.broadcasted_iota(jnp.int32, (TN, N), 1)
  sels = []
  for _ in range(K):
    m = jnp.min(acc, axis=1, keepdims=True)
    eq = acc == m
    sel = jnp.min(jnp.where(eq, iota, N), axis=1, keepdims=True)
    sels.append(sel)
    acc = jnp.where(iota == sel, jnp.float32(jnp.inf), acc)
  idx_ref[0] = jnp.concatenate(sels, axis=1) + b * N


def _knn_call(xyz, pt, interpret=False):
  return pl.pallas_call(
      _knn_body,
      grid=(B, N // TN),
      in_specs=[
          pl.BlockSpec((1, TN, 3), lambda b, t: (b, t, 0)),
          pl.BlockSpec((1, 3, N), lambda b, t: (b, 0, 0)),
      ],
      out_specs=pl.BlockSpec((1, TN, K), lambda b, t: (b, t, 0)),
      out_shape=jax.ShapeDtypeStruct((B, N, K), jnp.int32),
      interpret=interpret,
  )(xyz, pt)


_NC = 2
_NS = 16
_NW = _NC * _NS
_P = 4
_PK = _P * K
_NP = BN // _NW
_CHUNKS = _NP // _P


def _sc_body(y_ref, idx_ref, ymin_ref, ymax_ref,
             idx_v, rows_a, rows_b, omin_v, omax_v, sem_a, sem_b):
  cid = lax.axis_index("c")
  sid = lax.axis_index("s")
  wid = sid * _NC + cid
  crow0 = wid * _CHUNKS
  pt0 = wid * _NP

  pltpu.sync_copy(idx_ref.at[pl.ds(crow0, _CHUNKS)], idx_v)

  def gather(c, rows_v, sem):
    return pltpu.async_copy(y_ref.at[idx_v.at[c]], rows_v, sem)

  def compute(rows_v, c):
    for p in range(_P):
      r0 = p * K

      def jbody(j, _):
        js = pl.ds(j * 16, 16)
        vmin = rows_v[r0, js]
        vmax = vmin
        for k in range(1, K):
          v = rows_v[r0 + k, js]
          vmin = jnp.minimum(vmin, v)
          vmax = jnp.maximum(vmax, v)
        omin_v[p, js] = vmin
        omax_v[p, js] = vmax
        return 0

      lax.fori_loop(0, H // 16, jbody, 0)
    dst = pl.ds(pt0 + c * _P, _P)
    pltpu.sync_copy(omin_v, ymin_ref.at[dst])
    pltpu.sync_copy(omax_v, ymax_ref.at[dst])

  gather(0, rows_a, sem_a)

  def loop_body(i, _):
    c = i * 2
    gather(c + 1, rows_b, sem_b)
    pltpu.make_async_copy(y_ref.at[idx_v.at[c]], rows_a, sem_a).wait()
    compute(rows_a, c)
    gather_next = c + 2
    gather(gather_next, rows_a, sem_a)
    pltpu.make_async_copy(y_ref.at[idx_v.at[c + 1]], rows_b, sem_b).wait()
    compute(rows_b, c + 1)
    return 0

  lax.fori_loop(0, _CHUNKS // 2 - 1, loop_body, 0)
  c = _CHUNKS - 2
  gather(c + 1, rows_b, sem_b)
  pltpu.make_async_copy(y_ref.at[idx_v.at[c]], rows_a, sem_a).wait()
  compute(rows_a, c)
  pltpu.make_async_copy(y_ref.at[idx_v.at[c + 1]], rows_b, sem_b).wait()
  compute(rows_b, c + 1)


def _minmax_sc(y2, idx2):
  mesh = plsc.VectorSubcoreMesh(core_axis_name="c", subcore_axis_name="s")
  f = pl.kernel(
      _sc_body,
      out_type=(
          jax.ShapeDtypeStruct((BN, H), jnp.float32),
          jax.ShapeDtypeStruct((BN, H), jnp.float32),
      ),
      mesh=mesh,
      scratch_types=[
          pltpu.VMEM((_CHUNKS, _PK), jnp.int32),
          pltpu.VMEM((_PK, H), jnp.float32),
          pltpu.VMEM((_PK, H), jnp.float32),
          pltpu.VMEM((_P, H), jnp.float32),
          pltpu.VMEM((_P, H), jnp.float32),
          pltpu.SemaphoreType.DMA,
          pltpu.SemaphoreType.DMA,
      ],
  )
  return f(y2, idx2)


TD = 1024


def _gelu(v):
  return 0.5 * v * (1.0 + lax.erf(v * 0.7071067811865476))


def _out_body(ymin_ref, ymax_ref, z_ref, w2_ref, b2_ref, out_ref):
  zv = z_ref[:]
  h = jnp.maximum(_gelu(ymin_ref[:] + zv), _gelu(ymax_ref[:] + zv))
  out_ref[:] = (
      jnp.dot(h, w2_ref[:], preferred_element_type=jnp.float32) + b2_ref[:])


def _out_call(ymin, ymax, z, w2, b2_2d, interpret=False):
  return pl.pallas_call(
      _out_body,
      grid=(BN // TD,),
      in_specs=[
          pl.BlockSpec((TD, H), lambda t: (t, 0)),
          pl.BlockSpec((TD, H), lambda t: (t, 0)),
          pl.BlockSpec((TD, H), lambda t: (t, 0)),
          pl.BlockSpec((H, C), lambda t: (0, 0)),
          pl.BlockSpec((1, C), lambda t: (0, 0)),
      ],
      out_specs=pl.BlockSpec((TD, C), lambda t: (t, 0)),
      out_shape=jax.ShapeDtypeStruct((BN, C), jnp.float32),
      interpret=interpret,
  )(ymin, ymax, z, w2, b2_2d)


@jax.jit
def kernel(x, center_xyz, W1, b1, W2, b2):
  pt = jnp.swapaxes(center_xyz, 1, 2)
  y2, z2 = _yz_call(x, W1, jnp.reshape(b1, (1, H)))
  idx = _knn_call(center_xyz, pt)
  idx2 = jnp.reshape(idx, (BN // _P, _PK))
  ymin, ymax = _minmax_sc(y2, idx2)
  out = _out_call(ymin, ymax, z2, W2, jnp.reshape(b2, (1, C)))
  return jnp.reshape(out, (B, N, C))


# speedup vs baseline: 11.5630x; 1.4063x over previous
import functools

import jax
import jax.numpy as jnp
from jax import lax
from jax.experimental import pallas as pl
from jax.experimental.pallas import tpu as pltpu
from jax.experimental.pallas import tpu_sc as plsc

B, N, C, K, H = 8, 2048, 128, 20, 256
BN = B * N


def _yz_body(x_ref, w1_ref, b1_ref, y_ref, z_ref):
  xb = x_ref[0]
  w1a = w1_ref[:C, :]
  w1b = w1_ref[C:, :]
  wcat = jnp.concatenate([w1a, w1b - w1a], axis=1)
  yz = jnp.dot(xb, wcat, preferred_element_type=jnp.float32)
  y_ref[:] = yz[:, :H]
  z_ref[:] = yz[:, H:] + b1_ref[:]


def _yz_call(x, w1, b1_2d, interpret=False):
  return pl.pallas_call(
      _yz_body,
      grid=(B,),
      in_specs=[
          pl.BlockSpec((1, N, C), lambda b: (b, 0, 0)),
          pl.BlockSpec((2 * C, H), lambda b: (0, 0)),
          pl.BlockSpec((1, H), lambda b: (0, 0)),
      ],
      out_specs=[
          pl.BlockSpec((N, H), lambda b: (b, 0)),
          pl.BlockSpec((N, H), lambda b: (b, 0)),
      ],
      out_shape=(
          jax.ShapeDtypeStruct((BN, H), jnp.float32),
          jax.ShapeDtypeStruct((BN, H), jnp.float32),
      ),
      interpret=interpret,
  )(x, w1, b1_2d)


TN = 256


def _knn_body(xyz_ref, pt_ref, idx_ref, *, b0):
  b = pl.program_id(0) + b0
  q = xyz_ref[0]
  p = pt_ref[0]
  dot = jnp.dot(q.astype(jnp.bfloat16), p.astype(jnp.bfloat16),
                preferred_element_type=jnp.float32)
  inner = -2.0 * dot
  xxq = (q[:, 0:1] * q[:, 0:1] + q[:, 1:2] * q[:, 1:2]
         + q[:, 2:3] * q[:, 2:3])
  xxp = (p[0:1, :] * p[0:1, :] + p[1:2, :] * p[1:2, :]
         + p[2:3, :] * p[2:3, :])
  acc = xxp - ((-xxq) - inner)
  iota_f = lax.broadcasted_iota(jnp.int32, (TN, N), 1).astype(jnp.float32)
  sels = []
  for _ in range(K):
    m = jnp.min(acc, axis=1, keepdims=True)
    eq = acc == m
    sel = jnp.min(jnp.where(eq, iota_f, jnp.float32(N)), axis=1,
                  keepdims=True)
    sels.append(sel)
    acc = jnp.where(iota_f == sel, jnp.float32(jnp.inf), acc)
  idx_f = jnp.concatenate(sels, axis=1)
  idx_ref[0] = idx_f.astype(jnp.int32) + b * N


def _knn_call(xyz, pt, b0=0, nb=B, interpret=False):
  return pl.pallas_call(
      functools.partial(_knn_body, b0=b0),
      grid=(nb, N // TN),
      in_specs=[
          pl.BlockSpec((1, TN, 3), lambda b, t: (b + b0, t, 0)),
          pl.BlockSpec((1, 3, N), lambda b, t: (b + b0, 0, 0)),
      ],
      out_specs=pl.BlockSpec((1, TN, K), lambda b, t: (b, t, 0)),
      out_shape=jax.ShapeDtypeStruct((nb, N, K), jnp.int32),
      interpret=interpret,
  )(xyz, pt)


_NC = 2
_NS = 16
_NW = _NC * _NS
_P = 4
_PK = _P * K
_NP = BN // _NW
_CHUNKS = _NP // _P


def _sc_body(y_ref, idx_ref, ymin_ref, ymax_ref,
             idx_v, rows_a, rows_b, omin_v, omax_v, sem_a, sem_b,
             *, chunks):
  cid = lax.axis_index("c")
  sid = lax.axis_index("s")
  wid = sid * _NC + cid
  crow0 = wid * chunks
  pt0 = wid * chunks * _P

  pltpu.sync_copy(idx_ref.at[pl.ds(crow0, chunks)], idx_v)

  def gather(c, rows_v, sem):
    return pltpu.async_copy(y_ref.at[idx_v.at[c]], rows_v, sem)

  def compute(rows_v, c):
    for p in range(_P):
      r0 = p * K

      def jbody(j, _):
        js = pl.ds(j * 16, 16)
        vmin = rows_v[r0, js]
        vmax = vmin
        for k in range(1, K):
          v = rows_v[r0 + k, js]
          vmin = jnp.minimum(vmin, v)
          vmax = jnp.maximum(vmax, v)
        omin_v[p, js] = vmin
        omax_v[p, js] = vmax
        return 0

      lax.fori_loop(0, H // 16, jbody, 0)
    dst = pl.ds(pt0 + c * _P, _P)
    pltpu.sync_copy(omin_v, ymin_ref.at[dst])
    pltpu.sync_copy(omax_v, ymax_ref.at[dst])

  gather(0, rows_a, sem_a)

  def loop_body(i, _):
    c = i * 2
    gather(c + 1, rows_b, sem_b)
    pltpu.make_async_copy(y_ref.at[idx_v.at[c]], rows_a, sem_a).wait()
    compute(rows_a, c)
    gather_next = c + 2
    gather(gather_next, rows_a, sem_a)
    pltpu.make_async_copy(y_ref.at[idx_v.at[c + 1]], rows_b, sem_b).wait()
    compute(rows_b, c + 1)
    return 0

  lax.fori_loop(0, chunks // 2 - 1, loop_body, 0)
  c = chunks - 2
  gather(c + 1, rows_b, sem_b)
  pltpu.make_async_copy(y_ref.at[idx_v.at[c]], rows_a, sem_a).wait()
  compute(rows_a, c)
  pltpu.make_async_copy(y_ref.at[idx_v.at[c + 1]], rows_b, sem_b).wait()
  compute(rows_b, c + 1)


def _minmax_sc(y2, idx2):
  npts = idx2.shape[0] * _P
  chunks = npts // (_NW * _P)
  mesh = plsc.VectorSubcoreMesh(core_axis_name="c", subcore_axis_name="s")
  f = pl.kernel(
      functools.partial(_sc_body, chunks=chunks),
      out_type=(
          jax.ShapeDtypeStruct((npts, H), jnp.float32),
          jax.ShapeDtypeStruct((npts, H), jnp.float32),
      ),
      mesh=mesh,
      scratch_types=[
          pltpu.VMEM((chunks, _PK), jnp.int32),
          pltpu.VMEM((_PK, H), jnp.float32),
          pltpu.VMEM((_PK, H), jnp.float32),
          pltpu.VMEM((_P, H), jnp.float32),
          pltpu.VMEM((_P, H), jnp.float32),
          pltpu.SemaphoreType.DMA,
          pltpu.SemaphoreType.DMA,
      ],
  )
  return f(y2, idx2)


TD = 1024


def _gelu(v):
  return 0.5 * v * (1.0 + lax.erf(v * 0.7071067811865476))


def _out_body(ymin_ref, ymax_ref, z_ref, w2_ref, b2_ref, out_ref):
  zv = z_ref[:]
  h = jnp.maximum(_gelu(ymin_ref[:] + zv), _gelu(ymax_ref[:] + zv))
  out_ref[:] = (
      jnp.dot(h, w2_ref[:], preferred_element_type=jnp.float32) + b2_ref[:])


def _out_call(ymin, ymax, z, w2, b2_2d, t0=0, interpret=False):
  rows = ymin.shape[0]
  return pl.pallas_call(
      _out_body,
      grid=(rows // TD,),
      in_specs=[
          pl.BlockSpec((TD, H), lambda t: (t, 0)),
          pl.BlockSpec((TD, H), lambda t: (t, 0)),
          pl.BlockSpec((TD, H), lambda t: (t + t0, 0)),
          pl.BlockSpec((H, C), lambda t: (0, 0)),
          pl.BlockSpec((1, C), lambda t: (0, 0)),
      ],
      out_specs=pl.BlockSpec((TD, C), lambda t: (t, 0)),
      out_shape=jax.ShapeDtypeStruct((rows, C), jnp.float32),
      interpret=interpret,
  )(ymin, ymax, z, w2, b2_2d)


@jax.jit
def kernel(x, center_xyz, W1, b1, W2, b2):
  pt = jnp.swapaxes(center_xyz, 1, 2)
  y2, z2 = _yz_call(x, W1, jnp.reshape(b1, (1, H)))
  b2_2d = jnp.reshape(b2, (1, C))
  nbh = B // 2
  bnh = nbh * N
  outs = []
  idxs = [_knn_call(center_xyz, pt, b0=h * nbh, nb=nbh) for h in range(2)]
  for h in range(2):
    idx2 = jnp.reshape(idxs[h], (bnh // _P, _PK))
    ymin, ymax = _minmax_sc(y2, idx2)
    outs.append(_out_call(ymin, ymax, z2, W2, b2_2d, t0=h * (bnh // TD)))
  return jnp.reshape(jnp.concatenate(outs, axis=0), (B, N, C))


# speedup vs baseline: 12.2158x; 1.0565x over previous
import functools

import jax
import jax.numpy as jnp
from jax import lax
from jax.experimental import pallas as pl
from jax.experimental.pallas import tpu as pltpu
from jax.experimental.pallas import tpu_sc as plsc

B, N, C, K, H = 8, 2048, 128, 20, 256
BN = B * N


def _yz_body(x_ref, w1_ref, b1_ref, y_ref, z_ref):
  xb = x_ref[0]
  w1a = w1_ref[:C, :]
  w1b = w1_ref[C:, :]
  wcat = jnp.concatenate([w1a, w1b - w1a], axis=1)
  yz = jnp.dot(xb, wcat, preferred_element_type=jnp.float32)
  y_ref[:] = yz[:, :H]
  z_ref[:] = yz[:, H:] + b1_ref[:]


def _yz_call(x, w1, b1_2d, interpret=False):
  return pl.pallas_call(
      _yz_body,
      grid=(B,),
      in_specs=[
          pl.BlockSpec((1, N, C), lambda b: (b, 0, 0)),
          pl.BlockSpec((2 * C, H), lambda b: (0, 0)),
          pl.BlockSpec((1, H), lambda b: (0, 0)),
      ],
      out_specs=[
          pl.BlockSpec((N, H), lambda b: (b, 0)),
          pl.BlockSpec((N, H), lambda b: (b, 0)),
      ],
      out_shape=(
          jax.ShapeDtypeStruct((BN, H), jnp.float32),
          jax.ShapeDtypeStruct((BN, H), jnp.float32),
      ),
      interpret=interpret,
  )(x, w1, b1_2d)


TN = 256


def _knn_body(xyz_ref, pt_ref, idx_ref, *, b0):
  b = pl.program_id(0) + b0
  q = xyz_ref[0]
  p = pt_ref[0]
  dot = jnp.dot(q.astype(jnp.bfloat16), p.astype(jnp.bfloat16),
                preferred_element_type=jnp.float32)
  inner = -2.0 * dot
  xxq = (q[:, 0:1] * q[:, 0:1] + q[:, 1:2] * q[:, 1:2]
         + q[:, 2:3] * q[:, 2:3])
  xxp = (p[0:1, :] * p[0:1, :] + p[1:2, :] * p[1:2, :]
         + p[2:3, :] * p[2:3, :])
  acc = xxp - ((-xxq) - inner)
  iota_f = lax.broadcasted_iota(jnp.int32, (TN, N), 1).astype(jnp.float32)
  sels = []
  for _ in range(K):
    m = jnp.min(acc, axis=1, keepdims=True)
    eq = acc == m
    sel = jnp.min(jnp.where(eq, iota_f, jnp.float32(N)), axis=1,
                  keepdims=True)
    sels.append(sel)
    acc = jnp.where(iota_f == sel, jnp.float32(jnp.inf), acc)
  idx_f = jnp.concatenate(sels, axis=1)
  idx_ref[0] = idx_f.astype(jnp.int32) + b * N


def _knn_call(xyz, pt, b0=0, nb=B, interpret=False):
  return pl.pallas_call(
      functools.partial(_knn_body, b0=b0),
      grid=(nb, N // TN),
      in_specs=[
          pl.BlockSpec((1, TN, 3), lambda b, t: (b + b0, t, 0)),
          pl.BlockSpec((1, 3, N), lambda b, t: (b + b0, 0, 0)),
      ],
      out_specs=pl.BlockSpec((1, TN, K), lambda b, t: (b, t, 0)),
      out_shape=jax.ShapeDtypeStruct((nb, N, K), jnp.int32),
      interpret=interpret,
  )(xyz, pt)


_NC = 2
_NS = 16
_NW = _NC * _NS
_P = 4
_PK = _P * K
_NP = BN // _NW
_CHUNKS = _NP // _P


def _sc_body(y_ref, idx_ref, ymin_ref, ymax_ref,
             idx_v, rows_a, rows_b, omin_v, omax_v, sem_a, sem_b,
             *, chunks):
  cid = lax.axis_index("c")
  sid = lax.axis_index("s")
  wid = sid * _NC + cid
  crow0 = wid * chunks
  pt0 = wid * chunks * _P

  pltpu.sync_copy(idx_ref.at[pl.ds(crow0, chunks)], idx_v)

  def gather(c, rows_v, sem):
    return pltpu.async_copy(y_ref.at[idx_v.at[c]], rows_v, sem)

  def compute(rows_v, c):
    for p in range(_P):
      r0 = p * K

      def jbody(j, _):
        js = pl.ds(j * 16, 16)
        vmin = rows_v[r0, js]
        vmax = vmin
        for k in range(1, K):
          v = rows_v[r0 + k, js]
          vmin = jnp.minimum(vmin, v)
          vmax = jnp.maximum(vmax, v)
        omin_v[p, js] = vmin
        omax_v[p, js] = vmax
        return 0

      lax.fori_loop(0, H // 16, jbody, 0)
    dst = pl.ds(pt0 + c * _P, _P)
    pltpu.sync_copy(omin_v, ymin_ref.at[dst])
    pltpu.sync_copy(omax_v, ymax_ref.at[dst])

  gather(0, rows_a, sem_a)

  def loop_body(i, _):
    c = i * 2
    gather(c + 1, rows_b, sem_b)
    pltpu.make_async_copy(y_ref.at[idx_v.at[c]], rows_a, sem_a).wait()
    compute(rows_a, c)
    gather_next = c + 2
    gather(gather_next, rows_a, sem_a)
    pltpu.make_async_copy(y_ref.at[idx_v.at[c + 1]], rows_b, sem_b).wait()
    compute(rows_b, c + 1)
    return 0

  lax.fori_loop(0, chunks // 2 - 1, loop_body, 0)
  c = chunks - 2
  gather(c + 1, rows_b, sem_b)
  pltpu.make_async_copy(y_ref.at[idx_v.at[c]], rows_a, sem_a).wait()
  compute(rows_a, c)
  pltpu.make_async_copy(y_ref.at[idx_v.at[c + 1]], rows_b, sem_b).wait()
  compute(rows_b, c + 1)


def _minmax_sc(y2, idx2):
  npts = idx2.shape[0] * _P
  chunks = npts // (_NW * _P)
  mesh = plsc.VectorSubcoreMesh(core_axis_name="c", subcore_axis_name="s")
  f = pl.kernel(
      functools.partial(_sc_body, chunks=chunks),
      out_type=(
          jax.ShapeDtypeStruct((npts, H), jnp.float32),
          jax.ShapeDtypeStruct((npts, H), jnp.float32),
      ),
      mesh=mesh,
      scratch_types=[
          pltpu.VMEM((chunks, _PK), jnp.int32),
          pltpu.VMEM((_PK, H), jnp.float32),
          pltpu.VMEM((_PK, H), jnp.float32),
          pltpu.VMEM((_P, H), jnp.float32),
          pltpu.VMEM((_P, H), jnp.float32),
          pltpu.SemaphoreType.DMA,
          pltpu.SemaphoreType.DMA,
      ],
  )
  return f(y2, idx2)


TD = 1024


def _gelu(v):
  return 0.5 * v * (1.0 + lax.erf(v * 0.7071067811865476))


def _out_body(ymin_ref, ymax_ref, z_ref, w2_ref, b2_ref, out_ref):
  zv = z_ref[:]
  h = jnp.maximum(_gelu(ymin_ref[:] + zv), _gelu(ymax_ref[:] + zv))
  out_ref[:] = (
      jnp.dot(h, w2_ref[:], preferred_element_type=jnp.float32) + b2_ref[:])


def _out_call(ymin, ymax, z, w2, b2_2d, t0=0, interpret=False):
  rows = ymin.shape[0]
  return pl.pallas_call(
      _out_body,
      grid=(rows // TD,),
      in_specs=[
          pl.BlockSpec((TD, H), lambda t: (t, 0)),
          pl.BlockSpec((TD, H), lambda t: (t, 0)),
          pl.BlockSpec((TD, H), lambda t: (t + t0, 0)),
          pl.BlockSpec((H, C), lambda t: (0, 0)),
          pl.BlockSpec((1, C), lambda t: (0, 0)),
      ],
      out_specs=pl.BlockSpec((TD, C), lambda t: (t, 0)),
      out_shape=jax.ShapeDtypeStruct((rows, C), jnp.float32),
      interpret=interpret,
  )(ymin, ymax, z, w2, b2_2d)


@jax.jit
def kernel(x, center_xyz, W1, b1, W2, b2):
  pt = jnp.swapaxes(center_xyz, 1, 2)
  y2, z2 = _yz_call(x, W1, jnp.reshape(b1, (1, H)))
  b2_2d = jnp.reshape(b2, (1, C))
  NSPLIT = 4
  nbh = B // NSPLIT
  bnh = nbh * N
  outs = []
  idxs = [_knn_call(center_xyz, pt, b0=h * nbh, nb=nbh) for h in range(NSPLIT)]
  for h in range(NSPLIT):
    idx2 = jnp.reshape(idxs[h], (bnh // _P, _PK))
    ymin, ymax = _minmax_sc(y2, idx2)
    outs.append(_out_call(ymin, ymax, z2, W2, b2_2d, t0=h * (bnh // TD)))
  return jnp.reshape(jnp.concatenate(outs, axis=0), (B, N, C))
